# baseline (speedup 1.0000x reference)
import functools

import jax
import jax.numpy as jnp
import numpy as np
from jax import lax
from jax.experimental import pallas as pl
from jax.experimental.pallas import tpu as pltpu
from jax.experimental.pallas import tpu_sc as plsc

_CLUSTERS_N = 2
_ITERATIONS = 10
_NB = 4
_L = 16


def _init_perm(b):
    keys = jax.random.split(jax.random.key(42), b)
    return jax.vmap(lambda k: jax.random.permutation(k, 512)[:_CLUSTERS_N])(keys)


@functools.lru_cache(maxsize=None)
def _init_indices(b):
    with jax.ensure_compile_time_eval():
        idx = np.asarray(jax.device_get(_init_perm(b))).astype(np.int32)
    out = np.zeros((b, 2 * _L), dtype=np.int32)
    out[:, :_L] = idx[:, 0:1]
    out[:, _L:] = idx[:, 1:2]
    return out


def _init_for(b):
    try:
        return jnp.asarray(_init_indices(b))
    except Exception:
        perm = _init_perm(b).astype(jnp.int32)
        return jnp.concatenate(
            [jnp.broadcast_to(perm[:, 0:1], (b, _L)),
             jnp.broadcast_to(perm[:, 1:2], (b, _L))], axis=1)


def _coords_kernel(feat_ref, r_ref, c_ref):
    feat = feat_ref[...]
    nb, h, w, c = feat.shape
    hw = h * w
    m1 = jnp.max(feat, axis=1)
    maxv = jnp.max(m1, axis=1)
    lin = (lax.broadcasted_iota(jnp.int32, feat.shape, 1) * w
           + lax.broadcasted_iota(jnp.int32, feat.shape, 2))
    hit = jnp.where(feat == maxv[:, None, None, :], lin, hw)
    idx = jnp.min(jnp.min(hit, axis=1), axis=1)
    idx_f = idx.astype(jnp.float32)
    r = jnp.floor(idx_f / float(w))
    r_ref[0] = r
    c_ref[0] = idx_f - r * float(w)


def _mask_kernel(mask_ref, feat_ref, out0_ref, out1_ref):
    feat = feat_ref[...]
    m1 = mask_ref[0]
    o1 = feat * m1[:, None, None, :]
    out1_ref[...] = o1
    out0_ref[...] = feat - o1


def _sc_kmeans_body(r_hbm, c_hbm, init_hbm, mask_hbm, r_v, c_v, init_v, mask_v,
                    red_v):
    w = lax.axis_index("s") * 2 + lax.axis_index("c")
    pltpu.sync_copy(r_hbm.at[w], r_v)
    pltpu.sync_copy(c_hbm.at[w], c_v)
    pltpu.sync_copy(init_hbm.at[w], init_v)

    def allsum(x):
        for hop in (8, 4, 2, 1):
            red_v[pl.ds(0, _L)] = x
            red_v[pl.ds(_L, _L)] = x
            x = red_v[pl.ds(0, _L)] + red_v[pl.ds(hop, _L)]
        return x

    i0 = init_v[pl.ds(0, _L)]
    i1 = init_v[pl.ds(_L, _L)]

    nslices = 512 // _L
    zero = jnp.zeros((_L,), jnp.float32)

    def tot_body(j, acc):
        sr, sc, y0, x0, y1, x1 = acc
        rv = r_v[pl.ds(j * _L, _L)]
        cv = c_v[pl.ds(j * _L, _L)]
        lane = lax.broadcasted_iota(jnp.int32, (_L,), 0) + j * _L
        sel0 = lane == i0
        sel1 = lane == i1
        return (sr + rv, sc + cv,
                y0 + jnp.where(sel0, rv, 0.0), x0 + jnp.where(sel0, cv, 0.0),
                y1 + jnp.where(sel1, rv, 0.0), x1 + jnp.where(sel1, cv, 0.0))

    sr_a, sc_a, y0_a, x0_a, y1_a, x1_a = lax.fori_loop(
        0, nslices, tot_body, (zero,) * 6)
    sum_r = allsum(sr_a)
    sum_c = allsum(sc_a)
    cy0 = allsum(y0_a)
    cx0 = allsum(x0_a)
    cy1 = allsum(y1_a)
    cx1 = allsum(x1_a)
    one = jnp.ones((_L,), jnp.float32)
    total = jnp.full((_L,), 512.0, jnp.float32)

    def assign_sums(cy0, cx0, cy1, cx1):
        def j_body(j, acc):
            n1a, r1a, c1a = acc
            rv = r_v[pl.ds(j * _L, _L)]
            cv = c_v[pl.ds(j * _L, _L)]
            dy0 = rv - cy0
            dx0 = cv - cx0
            dy1 = rv - cy1
            dx1 = cv - cx1
            d0 = dy0 * dy0 + dx0 * dx0
            d1 = dy1 * dy1 + dx1 * dx1
            mf = jnp.where(d1 < d0, 1.0, 0.0).astype(jnp.float32)
            return (n1a + mf, r1a + rv * mf, c1a + cv * mf)
        return lax.fori_loop(0, nslices, j_body, (zero, zero, zero))

    for _ in range(_ITERATIONS):
        n1a, r1a, c1a = assign_sums(cy0, cx0, cy1, cx1)
        n1 = allsum(n1a)
        sr1 = allsum(r1a)
        sc1 = allsum(c1a)
        n1g = jnp.maximum(n1, one)
        n0g = jnp.maximum(total - n1, one)
        cy1 = sr1 / n1g
        cx1 = sc1 / n1g
        cy0 = (sum_r - sr1) / n0g
        cx0 = (sum_c - sc1) / n0g

    def mask_body(j, carry):
        rv = r_v[pl.ds(j * _L, _L)]
        cv = c_v[pl.ds(j * _L, _L)]
        dy0 = rv - cy0
        dx0 = cv - cx0
        dy1 = rv - cy1
        dx1 = cv - cx1
        d0 = dy0 * dy0 + dx0 * dx0
        d1 = dy1 * dy1 + dx1 * dx1
        mask_v[pl.ds(j * _L, _L)] = jnp.where(d1 < d0, 1.0, 0.0).astype(jnp.float32)
        return carry

    lax.fori_loop(0, nslices, mask_body, 0)
    pltpu.sync_copy(mask_v, mask_hbm.at[w])


def _sc_kmeans(r, c, init):
    b, ch = r.shape
    mesh = plsc.VectorSubcoreMesh(core_axis_name="c", subcore_axis_name="s")
    fn = functools.partial(
        pl.kernel,
        out_type=jax.ShapeDtypeStruct((b, ch), jnp.float32),
        mesh=mesh,
        scratch_types=[
            pltpu.VMEM((ch,), jnp.float32),
            pltpu.VMEM((ch,), jnp.float32),
            pltpu.VMEM((2 * _L,), jnp.int32),
            pltpu.VMEM((ch,), jnp.float32),
            pltpu.VMEM((2 * _L,), jnp.float32),
        ],
    )(_sc_kmeans_body)
    return fn(r, c, init)


def kernel(feature_batch):
    b, h, w, c = feature_batch.shape
    init = _init_for(b)
    grid = b // _NB

    r3, c3 = pl.pallas_call(
        _coords_kernel,
        grid=(grid,),
        in_specs=[pl.BlockSpec((_NB, h, w, c), lambda i: (i, 0, 0, 0))],
        out_specs=[
            pl.BlockSpec((1, _NB, c), lambda i: (i, 0, 0)),
            pl.BlockSpec((1, _NB, c), lambda i: (i, 0, 0)),
        ],
        out_shape=[
            jax.ShapeDtypeStruct((grid, _NB, c), jnp.float32),
            jax.ShapeDtypeStruct((grid, _NB, c), jnp.float32),
        ],
    )(feature_batch)

    mask = _sc_kmeans(r3.reshape(b, c), c3.reshape(b, c), init)

    return pl.pallas_call(
        _mask_kernel,
        grid=(grid,),
        in_specs=[
            pl.BlockSpec((1, _NB, c), lambda i: (i, 0, 0)),
            pl.BlockSpec((_NB, h, w, c), lambda i: (i, 0, 0, 0)),
        ],
        out_specs=[
            pl.BlockSpec((_NB, h, w, c), lambda i: (i, 0, 0, 0)),
            pl.BlockSpec((_NB, h, w, c), lambda i: (i, 0, 0, 0)),
        ],
        out_shape=[
            jax.ShapeDtypeStruct((b, h, w, c), feature_batch.dtype),
            jax.ShapeDtypeStruct((b, h, w, c), feature_batch.dtype),
        ],
    )(mask.reshape(grid, _NB, c), feature_batch)


# speedup vs baseline: 1.0763x; 1.0763x over previous
import functools

import jax
import jax.numpy as jnp
import numpy as np
from jax import lax
from jax.experimental import pallas as pl
from jax.experimental.pallas import tpu as pltpu
from jax.experimental.pallas import tpu_sc as plsc

_CLUSTERS_N = 2
_ITERATIONS = 10
_NB = 16
_L = 16


def _init_perm(b):
    keys = jax.random.split(jax.random.key(42), b)
    return jax.vmap(lambda k: jax.random.permutation(k, 512)[:_CLUSTERS_N])(keys)


@functools.lru_cache(maxsize=None)
def _init_indices(b):
    with jax.ensure_compile_time_eval():
        idx = np.asarray(jax.device_get(_init_perm(b))).astype(np.int32)
    out = np.zeros((b, 2 * _L), dtype=np.int32)
    out[:, :_L] = idx[:, 0:1]
    out[:, _L:] = idx[:, 1:2]
    return out


def _init_for(b):
    try:
        return jnp.asarray(_init_indices(b))
    except Exception:
        perm = _init_perm(b).astype(jnp.int32)
        return jnp.concatenate(
            [jnp.broadcast_to(perm[:, 0:1], (b, _L)),
             jnp.broadcast_to(perm[:, 1:2], (b, _L))], axis=1)


def _coords_kernel(feat_ref, r_ref, c_ref):
    feat = feat_ref[...]
    nb, h, w, c = feat.shape
    hw = h * w
    m1 = jnp.max(feat, axis=1)
    maxv = jnp.max(m1, axis=1)
    lin = (lax.broadcasted_iota(jnp.int32, feat.shape, 1) * w
           + lax.broadcasted_iota(jnp.int32, feat.shape, 2))
    hit = jnp.where(feat == maxv[:, None, None, :], lin, hw)
    idx = jnp.min(jnp.min(hit, axis=1), axis=1)
    idx_f = idx.astype(jnp.float32)
    r = jnp.floor(idx_f / float(w))
    r_ref[0] = r
    c_ref[0] = idx_f - r * float(w)


def _mask_kernel(mask_ref, feat_ref, out0_ref, out1_ref):
    feat = feat_ref[...]
    m1 = mask_ref[0]
    o1 = feat * m1[:, None, None, :]
    out1_ref[...] = o1
    out0_ref[...] = feat - o1


def _sc_kmeans_body(r_hbm, c_hbm, init_hbm, mask_hbm, r_v, c_v, init_v, mask_v,
                    red_v):
    w = lax.axis_index("s") * 2 + lax.axis_index("c")
    pltpu.sync_copy(r_hbm.at[w], r_v)
    pltpu.sync_copy(c_hbm.at[w], c_v)
    pltpu.sync_copy(init_hbm.at[w], init_v)

    def allsum(x):
        for hop in (8, 4, 2, 1):
            red_v[pl.ds(0, _L)] = x
            red_v[pl.ds(_L, _L)] = x
            x = red_v[pl.ds(0, _L)] + red_v[pl.ds(hop, _L)]
        return x

    i0 = init_v[pl.ds(0, _L)]
    i1 = init_v[pl.ds(_L, _L)]

    nslices = 512 // _L
    zero = jnp.zeros((_L,), jnp.float32)

    def tot_body(j, acc):
        sr, sc, y0, x0, y1, x1 = acc
        rv = r_v[pl.ds(j * _L, _L)]
        cv = c_v[pl.ds(j * _L, _L)]
        lane = lax.broadcasted_iota(jnp.int32, (_L,), 0) + j * _L
        sel0 = lane == i0
        sel1 = lane == i1
        return (sr + rv, sc + cv,
                y0 + jnp.where(sel0, rv, 0.0), x0 + jnp.where(sel0, cv, 0.0),
                y1 + jnp.where(sel1, rv, 0.0), x1 + jnp.where(sel1, cv, 0.0))

    sr_a, sc_a, y0_a, x0_a, y1_a, x1_a = lax.fori_loop(
        0, nslices, tot_body, (zero,) * 6)
    sum_r = allsum(sr_a)
    sum_c = allsum(sc_a)
    cy0 = allsum(y0_a)
    cx0 = allsum(x0_a)
    cy1 = allsum(y1_a)
    cx1 = allsum(x1_a)
    one = jnp.ones((_L,), jnp.float32)
    total = jnp.full((_L,), 512.0, jnp.float32)

    def assign_sums(cy0, cx0, cy1, cx1):
        def j_body(j, acc):
            n1a, r1a, c1a = acc
            rv = r_v[pl.ds(j * _L, _L)]
            cv = c_v[pl.ds(j * _L, _L)]
            dy0 = rv - cy0
            dx0 = cv - cx0
            dy1 = rv - cy1
            dx1 = cv - cx1
            d0 = dy0 * dy0 + dx0 * dx0
            d1 = dy1 * dy1 + dx1 * dx1
            mf = jnp.where(d1 < d0, 1.0, 0.0).astype(jnp.float32)
            return (n1a + mf, r1a + rv * mf, c1a + cv * mf)
        return lax.fori_loop(0, nslices, j_body, (zero, zero, zero))

    for _ in range(_ITERATIONS):
        n1a, r1a, c1a = assign_sums(cy0, cx0, cy1, cx1)
        n1 = allsum(n1a)
        sr1 = allsum(r1a)
        sc1 = allsum(c1a)
        n1g = jnp.maximum(n1, one)
        n0g = jnp.maximum(total - n1, one)
        cy1 = sr1 / n1g
        cx1 = sc1 / n1g
        cy0 = (sum_r - sr1) / n0g
        cx0 = (sum_c - sc1) / n0g

    def mask_body(j, carry):
        rv = r_v[pl.ds(j * _L, _L)]
        cv = c_v[pl.ds(j * _L, _L)]
        dy0 = rv - cy0
        dx0 = cv - cx0
        dy1 = rv - cy1
        dx1 = cv - cx1
        d0 = dy0 * dy0 + dx0 * dx0
        d1 = dy1 * dy1 + dx1 * dx1
        mask_v[pl.ds(j * _L, _L)] = jnp.where(d1 < d0, 1.0, 0.0).astype(jnp.float32)
        return carry

    lax.fori_loop(0, nslices, mask_body, 0)
    pltpu.sync_copy(mask_v, mask_hbm.at[w])


def _sc_kmeans(r, c, init):
    b, ch = r.shape
    mesh = plsc.VectorSubcoreMesh(core_axis_name="c", subcore_axis_name="s")
    fn = functools.partial(
        pl.kernel,
        out_type=jax.ShapeDtypeStruct((b, ch), jnp.float32),
        mesh=mesh,
        scratch_types=[
            pltpu.VMEM((ch,), jnp.float32),
            pltpu.VMEM((ch,), jnp.float32),
            pltpu.VMEM((2 * _L,), jnp.int32),
            pltpu.VMEM((ch,), jnp.float32),
            pltpu.VMEM((2 * _L,), jnp.float32),
        ],
    )(_sc_kmeans_body)
    return fn(r, c, init)


def kernel(feature_batch):
    b, h, w, c = feature_batch.shape
    init = _init_for(b)
    grid = b // _NB

    r3, c3 = pl.pallas_call(
        _coords_kernel,
        grid=(grid,),
        in_specs=[pl.BlockSpec((_NB, h, w, c), lambda i: (i, 0, 0, 0))],
        out_specs=[
            pl.BlockSpec((1, _NB, c), lambda i: (i, 0, 0)),
            pl.BlockSpec((1, _NB, c), lambda i: (i, 0, 0)),
        ],
        out_shape=[
            jax.ShapeDtypeStruct((grid, _NB, c), jnp.float32),
            jax.ShapeDtypeStruct((grid, _NB, c), jnp.float32),
        ],
    )(feature_batch)

    mask = _sc_kmeans(r3.reshape(b, c), c3.reshape(b, c), init)

    return pl.pallas_call(
        _mask_kernel,
        grid=(grid,),
        in_specs=[
            pl.BlockSpec((1, _NB, c), lambda i: (i, 0, 0)),
            pl.BlockSpec((_NB, h, w, c), lambda i: (i, 0, 0, 0)),
        ],
        out_specs=[
            pl.BlockSpec((_NB, h, w, c), lambda i: (i, 0, 0, 0)),
            pl.BlockSpec((_NB, h, w, c), lambda i: (i, 0, 0, 0)),
        ],
        out_shape=[
            jax.ShapeDtypeStruct((b, h, w, c), feature_batch.dtype),
            jax.ShapeDtypeStruct((b, h, w, c), feature_batch.dtype),
        ],
    )(mask.reshape(grid, _NB, c), feature_batch)
